# baseline (speedup 1.0000x reference)
import jax
import jax.numpy as jnp
from jax import lax
from jax.experimental import pallas as pl
from jax.experimental.pallas import tpu as pltpu
from jax.experimental.pallas import tpu_sc as plsc

_V = 1000000
_D = 64
_B = 16384 * 50
_PPB = 20
_CPB = 256 * _PPB
_NBLK = -(-_V // _CPB)
_PROWS = _NBLK * _PPB * 128
_TAIL = 999936

_info = plsc.get_sparse_core_info()
_NC, _NS = _info.num_cores, _info.num_subcores
_NW = _NC * _NS
_B_PER_W = _B // _NW
_CHUNK = 640
_GATHER = 128
_N_CHUNKS = _B_PER_W // _CHUNK
_NBUF = 2


def _pack_body(in_ref, out_ref):
    eye = jnp.float32(1) * (
        lax.broadcasted_iota(jnp.int32, (_D, _D), 0)
        == lax.broadcasted_iota(jnp.int32, (_D, _D), 1))
    at = lax.dot_general(in_ref[...], eye, (((0,), (0,)), ((), ())),
                         precision=lax.Precision.HIGHEST,
                         preferred_element_type=jnp.float32)
    for j in range(_PPB):
        out_ref[pl.ds(j * 128, 128), :] = jnp.concatenate(
            [at[j * 256:j * 256 + 128, :], at[j * 256 + 128:j * 256 + 256, :]],
            axis=1)


@jax.jit
def _pack(wt):
    return pl.pallas_call(
        _pack_body,
        grid=(_NBLK,),
        in_specs=[pl.BlockSpec((_D, _CPB), lambda i: (0, i))],
        out_specs=pl.BlockSpec((_PPB * 128, 128), lambda i: (i, 0)),
        out_shape=jax.ShapeDtypeStruct((_PROWS, 128), jnp.float32),
    )(wt)


def _gather_body(table_hbm, idx_hbm, out_hbm,
                 idx0, idx1, idxj0, idxj1, rows0, rows1,
                 isem0, isem1, gsem0, gsem1, osem0, osem1):
    wid = lax.axis_index("s") * _NC + lax.axis_index("c")
    base = wid * _B_PER_W
    idx_v = (idx0, idx1)
    idxj_v = (idxj0, idxj1)
    rows_v = (rows0, rows1)
    isem = (isem0, isem1)
    gsem = (gsem0, gsem1)
    osem = (osem0, osem1)

    def start_idx(b, i):
        pltpu.async_copy(
            idx_hbm.at[pl.ds(base + i * _CHUNK, _CHUNK)], idx_v[b], isem[b])

    def wait_idx(b):
        pltpu.make_async_copy(
            idx_hbm.at[pl.ds(0, _CHUNK)], idx_v[b], isem[b]).wait()

    def gather(b):
        for v in range(_CHUNK // 16):
            r = idx_v[b][pl.ds(v * 16, 16)]
            j = ((r >> 8) << 8) + ((r & 127) << 1) + ((r >> 7) & 1)
            j = jnp.where(r >= _TAIL, r * 2 - 999936, j)
            idxj_v[b][pl.ds(v * 16, 16)] = j
        for g in range(_CHUNK // _GATHER):
            pltpu.async_copy(
                table_hbm.at[idxj_v[b].at[pl.ds(g * _GATHER, _GATHER)]],
                rows_v[b].at[pl.ds(g * _GATHER, _GATHER)], gsem[b])
        for g in range(_CHUNK // _GATHER):
            pltpu.make_async_copy(
                table_hbm.at[idxj_v[b].at[pl.ds(0, _GATHER)]],
                rows_v[b].at[pl.ds(0, _GATHER)], gsem[b]).wait()

    def start_out(b, i):
        pltpu.async_copy(
            rows_v[b], out_hbm.at[pl.ds(base + i * _CHUNK, _CHUNK)], osem[b])

    def wait_out(b):
        pltpu.make_async_copy(
            rows_v[b], out_hbm.at[pl.ds(0, _CHUNK)], osem[b]).wait()

    for b in range(_NBUF):
        start_idx(b, b)
    for b in range(_NBUF):
        wait_idx(b)
        gather(b)
        start_out(b, b)
        start_idx(b, b + _NBUF)

    def step(g, carry):
        for b in range(_NBUF):
            i = g * _NBUF + b
            wait_idx(b)
            wait_out(b)
            gather(b)
            start_out(b, i)
            start_idx(b, i + _NBUF)
        return carry

    lax.fori_loop(1, _N_CHUNKS // _NBUF - 1, step, 0, unroll=False)

    for b in range(_NBUF):
        wait_idx(b)
        wait_out(b)
        gather(b)
        start_out(b, _N_CHUNKS - _NBUF + b)
    for b in range(_NBUF):
        wait_out(b)


@jax.jit
def _lookup(w64, idx):
    mesh = plsc.VectorSubcoreMesh(core_axis_name="c", subcore_axis_name="s")
    f = pl.kernel(
        _gather_body,
        mesh=mesh,
        out_type=jax.ShapeDtypeStruct((_B, _D), jnp.float32),
        scratch_types=[
            pltpu.VMEM((_CHUNK,), jnp.int32),
            pltpu.VMEM((_CHUNK,), jnp.int32),
            pltpu.VMEM((_CHUNK,), jnp.int32),
            pltpu.VMEM((_CHUNK,), jnp.int32),
            pltpu.VMEM((_CHUNK, _D), jnp.float32),
            pltpu.VMEM((_CHUNK, _D), jnp.float32),
            pltpu.SemaphoreType.DMA,
            pltpu.SemaphoreType.DMA,
            pltpu.SemaphoreType.DMA,
            pltpu.SemaphoreType.DMA,
            pltpu.SemaphoreType.DMA,
            pltpu.SemaphoreType.DMA,
        ],
        compiler_params=pltpu.CompilerParams(use_tc_tiling_on_sc=False),
    )
    return f(w64, idx)


def kernel(input_, weight):
    wt = weight.T
    packed = _pack(wt)
    w64 = packed.reshape(-1, _D)
    idx = input_.reshape(-1).astype(jnp.int32)
    out = _lookup(w64, idx)
    return out.reshape(input_.shape + (weight.shape[-1],))


# speedup vs baseline: 1.0351x; 1.0351x over previous
import jax
import jax.numpy as jnp
from jax import lax
from jax.experimental import pallas as pl
from jax.experimental.pallas import tpu as pltpu
from jax.experimental.pallas import tpu_sc as plsc

_V = 1000000
_D = 64
_B = 16384 * 50
_PPB = 40
_CPB = 256 * _PPB
_NBLK = -(-_V // _CPB)
_PROWS = _NBLK * _PPB * 128
_TAIL = 999936

_info = plsc.get_sparse_core_info()
_NC, _NS = _info.num_cores, _info.num_subcores
_NW = _NC * _NS
_B_PER_W = _B // _NW
_CHUNK = 640
_GATHER = 128
_N_CHUNKS = _B_PER_W // _CHUNK
_NBUF = 2


def _pack_body(in_ref, out_ref):
    eye = jnp.float32(1) * (
        lax.broadcasted_iota(jnp.int32, (_D, _D), 0)
        == lax.broadcasted_iota(jnp.int32, (_D, _D), 1))
    at = lax.dot_general(in_ref[...], eye, (((0,), (0,)), ((), ())),
                         precision=lax.Precision.HIGHEST,
                         preferred_element_type=jnp.float32)
    for j in range(_PPB):
        out_ref[pl.ds(j * 128, 128), :] = jnp.concatenate(
            [at[j * 256:j * 256 + 128, :], at[j * 256 + 128:j * 256 + 256, :]],
            axis=1)


@jax.jit
def _pack(wt):
    return pl.pallas_call(
        _pack_body,
        grid=(_NBLK,),
        in_specs=[pl.BlockSpec((_D, _CPB), lambda i: (0, i))],
        out_specs=pl.BlockSpec((_PPB * 128, 128), lambda i: (i, 0)),
        out_shape=jax.ShapeDtypeStruct((_PROWS, 128), jnp.float32),
    )(wt)


def _gather_body(table_hbm, idx_hbm, out_hbm,
                 idx0, idx1, idxj0, idxj1, rows0, rows1,
                 isem0, isem1, gsem0, gsem1, osem0, osem1):
    wid = lax.axis_index("s") * _NC + lax.axis_index("c")
    base = wid * _B_PER_W
    idx_v = (idx0, idx1)
    idxj_v = (idxj0, idxj1)
    rows_v = (rows0, rows1)
    isem = (isem0, isem1)
    gsem = (gsem0, gsem1)
    osem = (osem0, osem1)

    def start_idx(b, i):
        pltpu.async_copy(
            idx_hbm.at[pl.ds(base + i * _CHUNK, _CHUNK)], idx_v[b], isem[b])

    def wait_idx(b):
        pltpu.make_async_copy(
            idx_hbm.at[pl.ds(0, _CHUNK)], idx_v[b], isem[b]).wait()

    def gather(b):
        for v in range(_CHUNK // 16):
            r = idx_v[b][pl.ds(v * 16, 16)]
            j = ((r >> 8) << 8) + ((r & 127) << 1) + ((r >> 7) & 1)
            j = jnp.where(r >= _TAIL, r * 2 - 999936, j)
            idxj_v[b][pl.ds(v * 16, 16)] = j
        for g in range(_CHUNK // _GATHER):
            pltpu.async_copy(
                table_hbm.at[idxj_v[b].at[pl.ds(g * _GATHER, _GATHER)]],
                rows_v[b].at[pl.ds(g * _GATHER, _GATHER)], gsem[b])
        for g in range(_CHUNK // _GATHER):
            pltpu.make_async_copy(
                table_hbm.at[idxj_v[b].at[pl.ds(0, _GATHER)]],
                rows_v[b].at[pl.ds(0, _GATHER)], gsem[b]).wait()

    def start_out(b, i):
        pltpu.async_copy(
            rows_v[b], out_hbm.at[pl.ds(base + i * _CHUNK, _CHUNK)], osem[b])

    def wait_out(b):
        pltpu.make_async_copy(
            rows_v[b], out_hbm.at[pl.ds(0, _CHUNK)], osem[b]).wait()

    for b in range(_NBUF):
        start_idx(b, b)
    for b in range(_NBUF):
        wait_idx(b)
        gather(b)
        start_out(b, b)
        start_idx(b, b + _NBUF)

    def step(g, carry):
        for b in range(_NBUF):
            i = g * _NBUF + b
            wait_idx(b)
            wait_out(b)
            gather(b)
            start_out(b, i)
            start_idx(b, i + _NBUF)
        return carry

    lax.fori_loop(1, _N_CHUNKS // _NBUF - 1, step, 0, unroll=False)

    for b in range(_NBUF):
        wait_idx(b)
        wait_out(b)
        gather(b)
        start_out(b, _N_CHUNKS - _NBUF + b)
    for b in range(_NBUF):
        wait_out(b)


@jax.jit
def _lookup(w64, idx):
    mesh = plsc.VectorSubcoreMesh(core_axis_name="c", subcore_axis_name="s")
    f = pl.kernel(
        _gather_body,
        mesh=mesh,
        out_type=jax.ShapeDtypeStruct((_B, _D), jnp.float32),
        scratch_types=[
            pltpu.VMEM((_CHUNK,), jnp.int32),
            pltpu.VMEM((_CHUNK,), jnp.int32),
            pltpu.VMEM((_CHUNK,), jnp.int32),
            pltpu.VMEM((_CHUNK,), jnp.int32),
            pltpu.VMEM((_CHUNK, _D), jnp.float32),
            pltpu.VMEM((_CHUNK, _D), jnp.float32),
            pltpu.SemaphoreType.DMA,
            pltpu.SemaphoreType.DMA,
            pltpu.SemaphoreType.DMA,
            pltpu.SemaphoreType.DMA,
            pltpu.SemaphoreType.DMA,
            pltpu.SemaphoreType.DMA,
        ],
        compiler_params=pltpu.CompilerParams(use_tc_tiling_on_sc=False),
    )
    return f(w64, idx)


def kernel(input_, weight):
    wt = weight.T
    packed = _pack(wt)
    w64 = packed.reshape(-1, _D)
    idx = input_.reshape(-1).astype(jnp.int32)
    out = _lookup(w64, idx)
    return out.reshape(input_.shape + (weight.shape[-1],))
